# baseline (speedup 1.0000x reference)
import jax
import jax.numpy as jnp
from jax.experimental import pallas as pl
from jax.experimental.pallas import tpu as pltpu

_N, _P, _D = 4, 4096, 3
_QC = 2048
_NQ = _P // _QC


def _chamfer_kernel(src_ref, tgt_ref, out_ref, min_ref):
    b = pl.program_id(0)
    j = pl.program_id(1)

    S = src_ref[0]
    T = tgt_ref[0]

    x2 = jnp.sum(S * S, axis=1, keepdims=True)
    y2 = jnp.sum(T * T, axis=0, keepdims=True)

    xy = jax.lax.dot_general(
        S, T, (((1,), (0,)), ((), ())),
        preferred_element_type=jnp.float32,
    )
    d = x2 + y2 - 2.0 * xy
    m = jnp.min(d, axis=1, keepdims=True)

    @pl.when(j == 0)
    def _():
        min_ref[...] = m

    @pl.when(j > 0)
    def _():
        min_ref[...] = jnp.minimum(min_ref[...], m)

    @pl.when(jnp.logical_and(b == 0, j == 0))
    def _():
        out_ref[...] = jnp.zeros_like(out_ref)

    @pl.when(j == _NQ - 1)
    def _():
        s = jnp.sum(min_ref[...], keepdims=True) * (1.0 / _N)
        out_ref[...] += s


def kernel(source_cloud, target_cloud):
    tgt_t = target_cloud.transpose(0, 2, 1)
    out = pl.pallas_call(
        _chamfer_kernel,
        grid=(_N, _NQ),
        in_specs=[
            pl.BlockSpec((1, _P, _D), lambda b, j: (b, 0, 0)),
            pl.BlockSpec((1, _D, _QC), lambda b, j: (b, 0, j)),
        ],
        out_specs=pl.BlockSpec((1, 1), lambda b, j: (0, 0)),
        out_shape=jax.ShapeDtypeStruct((1, 1), jnp.float32),
        scratch_shapes=[pltpu.VMEM((_P, 1), jnp.float32)],
    )(source_cloud, tgt_t)
    return out[0, 0]


# speedup vs baseline: 1.0201x; 1.0201x over previous
import jax
import jax.numpy as jnp
from jax.experimental import pallas as pl
from jax.experimental.pallas import tpu as pltpu

_N, _P, _D = 4, 4096, 3
_QC = 2048
_NQ = _P // _QC


def _chamfer_kernel(src_ref, tgt_ref, out_ref, min_ref):
    b = pl.program_id(0)
    j = pl.program_id(1)

    S = src_ref[0]
    T = tgt_ref[0]

    x2 = jnp.sum(S * S, axis=1, keepdims=True)
    y2 = jnp.sum(T * T, axis=0, keepdims=True)

    x2_hi = x2.astype(jnp.bfloat16).astype(jnp.float32)
    x2_lo = x2 - x2_hi
    y2_hi = y2.astype(jnp.bfloat16).astype(jnp.float32)
    y2_lo = y2 - y2_hi
    ones_p = jnp.ones((_P, 1), jnp.float32)
    ones_q = jnp.ones((1, _QC), jnp.float32)
    A = jnp.concatenate([S, x2_hi, x2_lo, ones_p, ones_p], axis=1)
    B = jnp.concatenate([-2.0 * T, ones_q, ones_q, y2_hi, y2_lo],
                        axis=0)

    d = jax.lax.dot_general(
        A, B, (((1,), (0,)), ((), ())),
        preferred_element_type=jnp.float32,
    )
    m = jnp.min(d, axis=1, keepdims=True)

    @pl.when(j == 0)
    def _():
        min_ref[...] = m

    @pl.when(j > 0)
    def _():
        min_ref[...] = jnp.minimum(min_ref[...], m)

    @pl.when(jnp.logical_and(b == 0, j == 0))
    def _():
        out_ref[...] = jnp.zeros_like(out_ref)

    @pl.when(j == _NQ - 1)
    def _():
        s = jnp.sum(min_ref[...], keepdims=True) * (1.0 / _N)
        out_ref[...] += s


def kernel(source_cloud, target_cloud):
    tgt_t = target_cloud.transpose(0, 2, 1)
    out = pl.pallas_call(
        _chamfer_kernel,
        grid=(_N, _NQ),
        in_specs=[
            pl.BlockSpec((1, _P, _D), lambda b, j: (b, 0, 0)),
            pl.BlockSpec((1, _D, _QC), lambda b, j: (b, 0, j)),
        ],
        out_specs=pl.BlockSpec((1, 1), lambda b, j: (0, 0)),
        out_shape=jax.ShapeDtypeStruct((1, 1), jnp.float32),
        scratch_shapes=[pltpu.VMEM((_P, 1), jnp.float32)],
    )(source_cloud, tgt_t)
    return out[0, 0]


# speedup vs baseline: 1.1036x; 1.0819x over previous
import jax
import jax.numpy as jnp
from jax.experimental import pallas as pl
from jax.experimental.pallas import tpu as pltpu

_N, _P, _D = 4, 4096, 3
_QC = 2048
_NQ = _P // _QC


def _chamfer_kernel(src_ref, tgt_ref, out_ref, min_ref):
    b = pl.program_id(0)
    j = pl.program_id(1)

    St = src_ref[0]
    T = tgt_ref[0]

    x2 = jnp.sum(St * St, axis=0, keepdims=True)
    y2 = jnp.sum(T * T, axis=1, keepdims=True)

    x2_hi = x2.astype(jnp.bfloat16).astype(jnp.float32)
    x2_lo = x2 - x2_hi
    y2_hi = y2.astype(jnp.bfloat16).astype(jnp.float32)
    y2_lo = y2 - y2_hi
    ones_p = jnp.ones((1, _P), jnp.float32)
    ones_q = jnp.ones((_QC, 1), jnp.float32)
    L = jnp.concatenate([T, y2_hi, y2_lo, ones_q, ones_q], axis=1)
    R = jnp.concatenate([-2.0 * St, ones_p, ones_p, x2_hi, x2_lo],
                        axis=0)

    d = jax.lax.dot_general(
        L, R, (((1,), (0,)), ((), ())),
        preferred_element_type=jnp.float32,
    )
    m = jnp.min(d, axis=0, keepdims=True)

    @pl.when(j == 0)
    def _():
        min_ref[...] = m

    @pl.when(j > 0)
    def _():
        min_ref[...] = jnp.minimum(min_ref[...], m)

    @pl.when(jnp.logical_and(b == 0, j == 0))
    def _():
        out_ref[...] = jnp.zeros_like(out_ref)

    @pl.when(j == _NQ - 1)
    def _():
        s = jnp.sum(min_ref[...], keepdims=True) * (1.0 / _N)
        out_ref[...] += s


def kernel(source_cloud, target_cloud):
    src_t = source_cloud.transpose(0, 2, 1)
    out = pl.pallas_call(
        _chamfer_kernel,
        grid=(_N, _NQ),
        in_specs=[
            pl.BlockSpec((1, _D, _P), lambda b, j: (b, 0, 0)),
            pl.BlockSpec((1, _QC, _D), lambda b, j: (b, j, 0)),
        ],
        out_specs=pl.BlockSpec((1, 1), lambda b, j: (0, 0)),
        out_shape=jax.ShapeDtypeStruct((1, 1), jnp.float32),
        scratch_shapes=[pltpu.VMEM((1, _P), jnp.float32)],
    )(src_t, target_cloud)
    return out[0, 0]
